# Initial kernel scaffold; baseline (speedup 1.0000x reference)
import jax
import jax.numpy as jnp
from jax.experimental import pallas as pl


def kernel(sent_encoder_embeds, edge_index, W1, b1, W2, b2):
    raise NotImplementedError("write your pallas kernel here")


# speedup vs baseline: 12.6889x; 12.6889x over previous
import functools

import jax
import jax.numpy as jnp
from jax import lax
from jax.experimental import pallas as pl
from jax.experimental.pallas import tpu as pltpu
from jax.experimental.pallas import tpu_sc as plsc

N_NODES = 10000
D = 128
NC = 2
NS = 16
NW = NC * NS
CHUNK = 128
NP = 10240
ROWS_PER_TILE = NP // NS
RCH = ROWS_PER_TILE // CHUNK
DEG_W = 16

_MESH = plsc.VectorSubcoreMesh(core_axis_name="c", subcore_axis_name="s")


def _sc_deg(dst_pad, n_chunks):
    @functools.partial(
        pl.kernel,
        out_type=jax.ShapeDtypeStruct((NC, NP, DEG_W), jnp.float32),
        mesh=_MESH,
        scratch_types=[
            pltpu.VMEM((CHUNK,), jnp.int32),
            pltpu.VMEM((CHUNK, DEG_W), jnp.float32),
            pltpu.VMEM((CHUNK, DEG_W), jnp.float32),
            pltpu.VMEM_SHARED((NP, DEG_W), jnp.float32),
        ],
    )
    def deg_kernel(dst_hbm, out_hbm, idx_v, ones_v, zeros_v, hist_sh):
        cid = lax.axis_index("c")
        sid = lax.axis_index("s")
        wid = cid * NS + sid

        def fill(r, _):
            ones_v[r, :] = jnp.full((DEG_W,), 1.0, jnp.float32)
            zeros_v[r, :] = jnp.zeros((DEG_W,), jnp.float32)
            return 0

        lax.fori_loop(0, CHUNK, fill, 0)
        for k in range(RCH):
            pltpu.sync_copy(zeros_v, hist_sh.at[pl.ds(sid * ROWS_PER_TILE + k * CHUNK, CHUNK)])
        plsc.subcore_barrier()

        base = wid * n_chunks * CHUNK

        def body(k, _):
            pltpu.sync_copy(dst_hbm.at[pl.ds(base + k * CHUNK, CHUNK)], idx_v)
            pltpu.sync_copy(ones_v, hist_sh.at[idx_v], add=True)
            return 0

        lax.fori_loop(0, n_chunks, body, 0)
        plsc.subcore_barrier()
        for k in range(RCH):
            r0 = sid * ROWS_PER_TILE + k * CHUNK
            pltpu.sync_copy(hist_sh.at[pl.ds(r0, CHUNK)], out_hbm.at[cid, pl.ds(r0, CHUNK)])

    return deg_kernel(dst_pad)


def _sc_agg(table, src_pad, dst_pad, n_chunks):
    @functools.partial(
        pl.kernel,
        out_type=jax.ShapeDtypeStruct((NC, NP, D), jnp.float32),
        mesh=_MESH,
        scratch_types=[
            pltpu.VMEM((CHUNK,), jnp.int32),
            pltpu.VMEM((CHUNK,), jnp.int32),
            pltpu.VMEM((CHUNK, D), jnp.float32),
            pltpu.VMEM((CHUNK, D), jnp.float32),
            pltpu.VMEM_SHARED((NP, D), jnp.float32),
            pltpu.SemaphoreType.DMA,
        ],
    )
    def agg_kernel(x_hbm, src_hbm, dst_hbm, out_hbm, si_v, di_v, rows_v, zeros_v, acc_sh, sem):
        cid = lax.axis_index("c")
        sid = lax.axis_index("s")
        wid = cid * NS + sid

        def fill(r, _):
            def fill_c(c, _):
                zeros_v[r, pl.ds(c * 16, 16)] = jnp.zeros((16,), jnp.float32)
                return 0

            lax.fori_loop(0, D // 16, fill_c, 0)
            return 0

        lax.fori_loop(0, CHUNK, fill, 0)
        for k in range(RCH):
            pltpu.sync_copy(zeros_v, acc_sh.at[pl.ds(sid * ROWS_PER_TILE + k * CHUNK, CHUNK)])
        plsc.subcore_barrier()

        base = wid * n_chunks * CHUNK

        def body(k, _):
            pltpu.sync_copy(src_hbm.at[pl.ds(base + k * CHUNK, CHUNK)], si_v)
            pltpu.sync_copy(dst_hbm.at[pl.ds(base + k * CHUNK, CHUNK)], di_v)
            pltpu.async_copy(x_hbm.at[si_v], rows_v, sem).wait()
            pltpu.sync_copy(rows_v, acc_sh.at[di_v], add=True)
            return 0

        lax.fori_loop(0, n_chunks, body, 0)
        plsc.subcore_barrier()
        for k in range(RCH):
            r0 = sid * ROWS_PER_TILE + k * CHUNK
            pltpu.sync_copy(acc_sh.at[pl.ds(r0, CHUNK)], out_hbm.at[cid, pl.ds(r0, CHUNK)])

    return agg_kernel(table, src_pad, dst_pad)


_BR = 1024


def _dinv_block(degp_blk):
    deg = degp_blk[0, :, 0:1] + degp_blk[1, :, 0:1] + 1.0
    return lax.rsqrt(deg)


def _tc_prep(degp, x_pad):
    def body(degp_ref, x_ref, xs_ref):
        xs_ref[...] = x_ref[...] * _dinv_block(degp_ref[...])

    return pl.pallas_call(
        body,
        grid=(NP // _BR,),
        in_specs=[
            pl.BlockSpec((NC, _BR, DEG_W), lambda i: (0, i, 0)),
            pl.BlockSpec((_BR, D), lambda i: (i, 0)),
        ],
        out_specs=pl.BlockSpec((_BR, D), lambda i: (i, 0)),
        out_shape=jax.ShapeDtypeStruct((NP, D), jnp.float32),
    )(degp, x_pad)


def _tc_mid(p, xs, degp, W1, b1, W2):
    def body(p_ref, xs_ref, degp_ref, w1_ref, b1_ref, w2_ref, hs_ref):
        i = pl.program_id(0)
        dinv = _dinv_block(degp_ref[...])
        y = (p_ref[0] + p_ref[1] + xs_ref[...]) * dinv
        x1 = jnp.maximum(jnp.dot(y, w1_ref[...], preferred_element_type=jnp.float32) + b1_ref[...], 0.0)
        h = jnp.dot(x1, w2_ref[...], preferred_element_type=jnp.float32)
        row = i * _BR + lax.broadcasted_iota(jnp.int32, (_BR, 1), 0)
        hs_ref[...] = jnp.where(row < N_NODES, h * dinv, 0.0)

    return pl.pallas_call(
        body,
        grid=(NP // _BR,),
        in_specs=[
            pl.BlockSpec((NC, _BR, D), lambda i: (0, i, 0)),
            pl.BlockSpec((_BR, D), lambda i: (i, 0)),
            pl.BlockSpec((NC, _BR, DEG_W), lambda i: (0, i, 0)),
            pl.BlockSpec((D, 2 * D), lambda i: (0, 0)),
            pl.BlockSpec((1, 2 * D), lambda i: (0, 0)),
            pl.BlockSpec((2 * D, D), lambda i: (0, 0)),
        ],
        out_specs=pl.BlockSpec((_BR, D), lambda i: (i, 0)),
        out_shape=jax.ShapeDtypeStruct((NP, D), jnp.float32),
    )(p, xs, degp, W1, b1, W2)


def _tc_final(q, hs, degp, b2):
    def body(q_ref, hs_ref, degp_ref, b2_ref, out_ref):
        dinv = _dinv_block(degp_ref[...])
        out_ref[...] = (q_ref[0] + q_ref[1] + hs_ref[...]) * dinv + b2_ref[...]

    return pl.pallas_call(
        body,
        grid=(NP // _BR,),
        in_specs=[
            pl.BlockSpec((NC, _BR, D), lambda i: (0, i, 0)),
            pl.BlockSpec((_BR, D), lambda i: (i, 0)),
            pl.BlockSpec((NC, _BR, DEG_W), lambda i: (0, i, 0)),
            pl.BlockSpec((1, D), lambda i: (0, 0)),
        ],
        out_specs=pl.BlockSpec((_BR, D), lambda i: (i, 0)),
        out_shape=jax.ShapeDtypeStruct((NP, D), jnp.float32),
    )(q, hs, degp, b2)


def kernel(sent_encoder_embeds, edge_index, W1, b1, W2, b2):
    x = sent_encoder_embeds
    src = edge_index[0]
    dst = edge_index[1]
    n_edges = src.shape[0]
    per_w = NW * CHUNK
    ep = ((n_edges + per_w - 1) // per_w) * per_w
    n_chunks = ep // per_w
    pad = jnp.full((ep - n_edges,), N_NODES, jnp.int32)
    src_p = jnp.concatenate([src, pad])
    dst_p = jnp.concatenate([dst, pad])
    x_pad = jnp.pad(x, ((0, NP - N_NODES), (0, 0)))

    degp = _sc_deg(dst_p, n_chunks)
    xs = _tc_prep(degp, x_pad)
    p = _sc_agg(xs, src_p, dst_p, n_chunks)
    hs = _tc_mid(p, xs, degp, W1.astype(jnp.float32), b1.reshape(1, -1), W2.astype(jnp.float32))
    q = _sc_agg(hs, src_p, dst_p, n_chunks)
    out = _tc_final(q, hs, degp, b2.reshape(1, -1))
    return out[:N_NODES]
